# baseline (speedup 1.0000x reference)
import dataclasses
import functools

import jax
import jax.numpy as jnp
from jax import lax
from jax.experimental import pallas as pl
from jax.experimental.pallas import tpu as pltpu
from jax.experimental.pallas import tpu_sc as plsc

B, S, D, E, TOP_K = 4, 2048, 1024, 8, 2
D_FF = 4 * D
N = B * S
BLK = 128
BLK_SH = 7
NW = 32
CHUNK = N // NW
NBLK = N // BLK + E
NPAD = NBLK * BLK
NMAP = ((NBLK + 16) // 16) * 16
SUB = 64
NSUB = CHUNK // SUB


BLKR = 1024
NCH_R = BLKR // CHUNK


def _router_body(x_ref, wg_ref, bg_ref, xw_ref, e_ref, cnt_ref):
    xb = x_ref[...]
    logits = jnp.dot(xb, wg_ref[...], preferred_element_type=jnp.float32)
    logits = logits + bg_ref[...]
    m = jnp.max(logits, axis=-1, keepdims=True)
    ex = jnp.exp(logits - m)
    s = ex / jnp.sum(ex, axis=-1, keepdims=True)
    lane = lax.broadcasted_iota(jnp.int32, (BLKR, E), 1)
    m1 = jnp.max(s, axis=-1, keepdims=True)
    i1 = jnp.min(jnp.where(s == m1, lane, E), axis=-1, keepdims=True)
    s2 = jnp.where(lane == i1, -jnp.inf, s)
    m2 = jnp.max(s2, axis=-1, keepdims=True)
    i2 = jnp.min(jnp.where(s2 == m2, lane, E), axis=-1, keepdims=True)
    estar = jnp.maximum(i1, i2)
    w = jnp.sum(jnp.where(lane == estar, s, 0.0), axis=-1, keepdims=True)
    xw_ref[...] = xb * w
    e_ref[...] = estar
    lane16 = lax.broadcasted_iota(jnp.int32, (BLKR, 16), 1)
    oh = (lane16 == estar).astype(jnp.int32)
    cnt_ref[...] = jnp.sum(oh.reshape(NCH_R, CHUNK, 16), axis=1,
                           keepdims=False).reshape(1, NCH_R, 16)


def _router(xf, Wg, bg):
    return pl.pallas_call(
        _router_body,
        grid=(N // BLKR,),
        in_specs=[
            pl.BlockSpec((BLKR, D), lambda b: (b, 0)),
            pl.BlockSpec((D, E), lambda b: (0, 0)),
            pl.BlockSpec((1, E), lambda b: (0, 0)),
        ],
        out_specs=[
            pl.BlockSpec((BLKR, D), lambda b: (b, 0)),
            pl.BlockSpec((BLKR, 1), lambda b: (b, 0)),
            pl.BlockSpec((1, NCH_R, 16), lambda b: (b, 0, 0)),
        ],
        out_shape=[
            jax.ShapeDtypeStruct((N, D), jnp.float32),
            jax.ShapeDtypeStruct((N, 1), jnp.int32),
            jax.ShapeDtypeStruct((N // BLKR, NCH_R, 16), jnp.int32),
        ],
    )(xf, Wg, bg.reshape(1, E))


_SC_MESH = plsc.VectorSubcoreMesh(
    core_axis_name="c", subcore_axis_name="s", num_cores=2, num_subcores=16)

_SC_PARAMS = pltpu.CompilerParams()
if "needs_layout_passes" in pltpu.CompilerParams.__dataclass_fields__:
    _SC_PARAMS = dataclasses.replace(_SC_PARAMS, needs_layout_passes=False)


@functools.partial(
    pl.kernel,
    out_type=[
        jax.ShapeDtypeStruct((NPAD, D), jnp.float32),
        jax.ShapeDtypeStruct((N,), jnp.int32),
        jax.ShapeDtypeStruct((NMAP,), jnp.int32),
    ],
    mesh=_SC_MESH,
    scratch_types=[
        pltpu.VMEM((NW, 16), jnp.int32),
        pltpu.VMEM((CHUNK,), jnp.int32),
        pltpu.VMEM((CHUNK,), jnp.int32),
        pltpu.VMEM((NSUB, SUB), jnp.int32),
        pltpu.VMEM((SUB, D), jnp.float32),
        pltpu.VMEM((NMAP,), jnp.int32),
        pltpu.SMEM((E,), jnp.int32),
        pltpu.SemaphoreType.DMA,
    ],
    compiler_params=_SC_PARAMS,
)
def _dispatch(e_hbm, cnt_hbm, xw_hbm, xs_hbm, dst_hbm, map_hbm,
              cnt_v, e_v, dst_v, idx_v, buf_v, map_v, base_s, sem):
    wid = lax.axis_index("s") * 2 + lax.axis_index("c")
    t0 = wid * CHUNK
    pltpu.sync_copy(cnt_hbm, cnt_v)
    pltpu.sync_copy(e_hbm.at[pl.ds(t0, CHUNK)], e_v)

    lane = lax.iota(jnp.int32, 16)
    total = jnp.zeros((16,), jnp.int32)
    pref = jnp.zeros((16,), jnp.int32)
    for wp in range(NW):
        row = cnt_v[wp]
        total = total + row
        pref = pref + jnp.where(wp < wid, row, 0)
    rounded = ((total + (BLK - 1)) >> BLK_SH) << BLK_SH
    rounded = jnp.where(lane < E, rounded, 0)
    incl = plsc.cumsum(rounded)
    seg_start = incl - rounded
    my_base = seg_start + pref

    for e in range(E):
        base_s[e] = jnp.sum(jnp.where(lane == e, my_base, 0))

    for k in range(CHUNK // 16):
        ev = e_v[pl.ds(k * 16, 16)]
        dstv = jnp.zeros((16,), jnp.int32)
        for e in range(E):
            mi = (ev == e).astype(jnp.int32)
            ranks = plsc.cumsum(mi) - 1
            b = base_s[e]
            dstv = jnp.where(ev == e, b + ranks, dstv)
            base_s[e] = b + jnp.sum(mi)
        dst_v[pl.ds(k * 16, 16)] = dstv
        idx_v[k // (SUB // 16), pl.ds((k % (SUB // 16)) * 16, 16)] = dstv

    pltpu.sync_copy(dst_v, dst_hbm.at[pl.ds(t0, CHUNK)])

    for j in range(NSUB):
        pltpu.sync_copy(xw_hbm.at[pl.ds(t0 + j * SUB, SUB)], buf_v)
        pltpu.async_copy(buf_v, xs_hbm.at[idx_v.at[j]], sem).wait()

    @pl.when(wid == 0)
    def _():
        nblk_used = jnp.sum(jnp.where(lane == E - 1, incl, 0)) >> BLK_SH
        for j in range(NMAP // 16):
            pos = (lax.iota(jnp.int32, 16) + j * 16) * BLK
            cnt = jnp.zeros((16,), jnp.int32)
            for e in range(1, E):
                st = jnp.sum(jnp.where(lane == e, seg_start, 0))
                cnt = cnt + (pos >= st).astype(jnp.int32)
            if j == NMAP // 16 - 1:
                cnt = jnp.where(lax.iota(jnp.int32, 16) == 15, nblk_used, cnt)
            map_v[pl.ds(j * 16, 16)] = cnt
        pltpu.sync_copy(map_v, map_hbm)


NQ = 4
QF = D_FF // NQ


def _ffn_body(map_ref, xs_ref, w1_hbm, b1_ref, w2_hbm, b2_ref, ys_ref,
              w1s, w2f, st, s1a, s1b, s2):
    b = pl.program_id(0)
    e = map_ref[b]
    nb = map_ref[NMAP - 1]
    live = b < nb
    is_sw = jnp.logical_and(live, jnp.logical_or(b == 0, e != st[0]))
    par = jnp.where(b == 0, 0, jnp.where(is_sw, 1 - st[1], st[1]))

    @pl.when(jnp.logical_and(is_sw, b == 0))
    def _():
        pltpu.make_async_copy(w1_hbm.at[e], w1s.at[0], s1a).start()

    @pl.when(is_sw)
    def _():
        for q in range(NQ):
            pltpu.make_async_copy(w2_hbm.at[e, pl.ds(q * QF, QF), :],
                                  w2f.at[pl.ds(q * QF, QF), :],
                                  s2.at[q]).start()

        @pl.when(par == 0)
        def _():
            pltpu.make_async_copy(w1_hbm.at[e], w1s.at[0], s1a).wait()

        @pl.when(par == 1)
        def _():
            pltpu.make_async_copy(w1_hbm.at[e], w1s.at[1], s1b).wait()

        st[0] = e
        st[1] = par
        st[2] = -1

    def _tail(w1buf):
        xb = xs_ref[...]
        y = jnp.zeros((BLK, D), jnp.float32)
        for q in range(NQ):
            hq = jnp.dot(xb, w1buf[:, q * QF:(q + 1) * QF],
                         preferred_element_type=jnp.float32)
            hq = jnp.maximum(hq + b1_ref[0, :, q * QF:(q + 1) * QF],
                             0.0).astype(jnp.bfloat16)

            @pl.when(is_sw)
            def _():
                pltpu.make_async_copy(w2_hbm.at[e, pl.ds(q * QF, QF), :],
                                      w2f.at[pl.ds(q * QF, QF), :],
                                      s2.at[q]).wait()

            y = y + jnp.dot(hq, w2f[q * QF:(q + 1) * QF, :],
                            preferred_element_type=jnp.float32)
        ys_ref[...] = y + b2_ref[0]

    @pl.when(live)
    def _():
        _tail(w1s.at[par])

    en1 = map_ref[b + 1]
    en2 = map_ref[b + 2]
    pe_next = jnp.where(en1 != e, en1, en2)
    vb = jnp.where(en1 != e, b + 1, b + 2)
    do_pref = live & (vb < nb) & (pe_next != e) & (st[2] != pe_next)

    @pl.when(do_pref)
    def _():
        @pl.when(par == 1)
        def _():
            pltpu.make_async_copy(w1_hbm.at[pe_next], w1s.at[0], s1a).start()

        @pl.when(par == 0)
        def _():
            pltpu.make_async_copy(w1_hbm.at[pe_next], w1s.at[1], s1b).start()

        st[2] = pe_next


def _ffn(bmap, xs, W1, b1, W2, b2):
    grid_spec = pltpu.PrefetchScalarGridSpec(
        num_scalar_prefetch=1,
        grid=(NBLK,),
        in_specs=[
            pl.BlockSpec((BLK, D), lambda b, m: (b, 0)),
            pl.BlockSpec(memory_space=pl.ANY),
            pl.BlockSpec((1, 1, D_FF), lambda b, m: (m[b], 0, 0)),
            pl.BlockSpec(memory_space=pl.ANY),
            pl.BlockSpec((1, 1, D), lambda b, m: (m[b], 0, 0)),
        ],
        out_specs=pl.BlockSpec((BLK, D), lambda b, m: (b, 0)),
        scratch_shapes=[
            pltpu.VMEM((2, D, D_FF), jnp.float32),
            pltpu.VMEM((D_FF, D), jnp.float32),
            pltpu.SMEM((4,), jnp.int32),
            pltpu.SemaphoreType.DMA,
            pltpu.SemaphoreType.DMA,
            pltpu.SemaphoreType.DMA((NQ,)),
        ],
    )
    return pl.pallas_call(
        _ffn_body,
        grid_spec=grid_spec,
        out_shape=jax.ShapeDtypeStruct((NPAD, D), jnp.float32),
        compiler_params=pltpu.CompilerParams(
            dimension_semantics=("arbitrary",)),
    )(bmap, xs, W1, b1, W2, b2)


@functools.partial(
    pl.kernel,
    out_type=jax.ShapeDtypeStruct((N, D), jnp.float32),
    mesh=_SC_MESH,
    scratch_types=[
        pltpu.VMEM((NSUB, SUB), jnp.int32),
        pltpu.VMEM((SUB, D), jnp.float32),
        pltpu.SemaphoreType.DMA,
    ],
    compiler_params=_SC_PARAMS,
)
def _combine(ys_hbm, dst_hbm, out_hbm, idx_v, buf_v, sem):
    wid = lax.axis_index("s") * 2 + lax.axis_index("c")
    t0 = wid * CHUNK
    for j in range(NSUB):
        pltpu.sync_copy(dst_hbm.at[pl.ds(t0 + j * SUB, SUB)], idx_v.at[j])
    for j in range(NSUB):
        pltpu.async_copy(ys_hbm.at[idx_v.at[j]], buf_v, sem).wait()
        pltpu.sync_copy(buf_v, out_hbm.at[pl.ds(t0 + j * SUB, SUB)])


def kernel(x, Wg, bg, W1, b1, W2, b2):
    xf = x.reshape(N, D)
    xw, e2, cnt3 = _router(xf, Wg, bg)
    xs, dst, bmap = _dispatch(e2.reshape(N), cnt3.reshape(NW, 16), xw)
    ys = _ffn(bmap, xs, W1, b1.reshape(E, 1, D_FF), W2, b2.reshape(E, 1, D))
    out = _combine(ys, dst)
    return out.reshape(B, S, D)


# speedup vs baseline: 1.0887x; 1.0887x over previous
import dataclasses
import functools

import jax
import jax.numpy as jnp
from jax import lax
from jax.experimental import pallas as pl
from jax.experimental.pallas import tpu as pltpu
from jax.experimental.pallas import tpu_sc as plsc

B, S, D, E, TOP_K = 4, 2048, 1024, 8, 2
D_FF = 4 * D
N = B * S
BLK = 256
BLK_SH = 8
NW = 32
CHUNK = N // NW
NBLK = N // BLK + E
NPAD = NBLK * BLK
NMAP = ((NBLK + 16) // 16) * 16
SUB = 64
NSUB = CHUNK // SUB


BLKR = 1024
NCH_R = BLKR // CHUNK


def _router_body(x_ref, wg_ref, bg_ref, xw_ref, e_ref, cnt_ref):
    xb = x_ref[...]
    logits = jnp.dot(xb, wg_ref[...], preferred_element_type=jnp.float32)
    logits = logits + bg_ref[...]
    m = jnp.max(logits, axis=-1, keepdims=True)
    ex = jnp.exp(logits - m)
    s = ex / jnp.sum(ex, axis=-1, keepdims=True)
    lane = lax.broadcasted_iota(jnp.int32, (BLKR, E), 1)
    m1 = jnp.max(s, axis=-1, keepdims=True)
    i1 = jnp.min(jnp.where(s == m1, lane, E), axis=-1, keepdims=True)
    s2 = jnp.where(lane == i1, -jnp.inf, s)
    m2 = jnp.max(s2, axis=-1, keepdims=True)
    i2 = jnp.min(jnp.where(s2 == m2, lane, E), axis=-1, keepdims=True)
    estar = jnp.maximum(i1, i2)
    w = jnp.sum(jnp.where(lane == estar, s, 0.0), axis=-1, keepdims=True)
    xw_ref[...] = xb * w
    e_ref[...] = estar
    lane16 = lax.broadcasted_iota(jnp.int32, (BLKR, 16), 1)
    oh = (lane16 == estar).astype(jnp.int32)
    cnt_ref[...] = jnp.sum(oh.reshape(NCH_R, CHUNK, 16), axis=1,
                           keepdims=False).reshape(1, NCH_R, 16)


def _router(xf, Wg, bg):
    return pl.pallas_call(
        _router_body,
        grid=(N // BLKR,),
        in_specs=[
            pl.BlockSpec((BLKR, D), lambda b: (b, 0)),
            pl.BlockSpec((D, E), lambda b: (0, 0)),
            pl.BlockSpec((1, E), lambda b: (0, 0)),
        ],
        out_specs=[
            pl.BlockSpec((BLKR, D), lambda b: (b, 0)),
            pl.BlockSpec((BLKR, 1), lambda b: (b, 0)),
            pl.BlockSpec((1, NCH_R, 16), lambda b: (b, 0, 0)),
        ],
        out_shape=[
            jax.ShapeDtypeStruct((N, D), jnp.float32),
            jax.ShapeDtypeStruct((N, 1), jnp.int32),
            jax.ShapeDtypeStruct((N // BLKR, NCH_R, 16), jnp.int32),
        ],
    )(xf, Wg, bg.reshape(1, E))


_SC_MESH = plsc.VectorSubcoreMesh(
    core_axis_name="c", subcore_axis_name="s", num_cores=2, num_subcores=16)

_SC_PARAMS = pltpu.CompilerParams()
if "needs_layout_passes" in pltpu.CompilerParams.__dataclass_fields__:
    _SC_PARAMS = dataclasses.replace(_SC_PARAMS, needs_layout_passes=False)


@functools.partial(
    pl.kernel,
    out_type=[
        jax.ShapeDtypeStruct((NPAD, D), jnp.float32),
        jax.ShapeDtypeStruct((N,), jnp.int32),
        jax.ShapeDtypeStruct((NMAP,), jnp.int32),
    ],
    mesh=_SC_MESH,
    scratch_types=[
        pltpu.VMEM((NW, 16), jnp.int32),
        pltpu.VMEM((CHUNK,), jnp.int32),
        pltpu.VMEM((CHUNK,), jnp.int32),
        pltpu.VMEM((NSUB, SUB), jnp.int32),
        pltpu.VMEM((SUB, D), jnp.float32),
        pltpu.VMEM((NMAP,), jnp.int32),
        pltpu.SMEM((E,), jnp.int32),
        pltpu.SemaphoreType.DMA,
    ],
    compiler_params=_SC_PARAMS,
)
def _dispatch(e_hbm, cnt_hbm, xw_hbm, xs_hbm, dst_hbm, map_hbm,
              cnt_v, e_v, dst_v, idx_v, buf_v, map_v, base_s, sem):
    wid = lax.axis_index("s") * 2 + lax.axis_index("c")
    t0 = wid * CHUNK
    pltpu.sync_copy(cnt_hbm, cnt_v)
    pltpu.sync_copy(e_hbm.at[pl.ds(t0, CHUNK)], e_v)

    lane = lax.iota(jnp.int32, 16)
    total = jnp.zeros((16,), jnp.int32)
    pref = jnp.zeros((16,), jnp.int32)
    for wp in range(NW):
        row = cnt_v[wp]
        total = total + row
        pref = pref + jnp.where(wp < wid, row, 0)
    rounded = ((total + (BLK - 1)) >> BLK_SH) << BLK_SH
    rounded = jnp.where(lane < E, rounded, 0)
    incl = plsc.cumsum(rounded)
    seg_start = incl - rounded
    my_base = seg_start + pref

    for e in range(E):
        base_s[e] = jnp.sum(jnp.where(lane == e, my_base, 0))

    for k in range(CHUNK // 16):
        ev = e_v[pl.ds(k * 16, 16)]
        dstv = jnp.zeros((16,), jnp.int32)
        for e in range(E):
            mi = (ev == e).astype(jnp.int32)
            ranks = plsc.cumsum(mi) - 1
            b = base_s[e]
            dstv = jnp.where(ev == e, b + ranks, dstv)
            base_s[e] = b + jnp.sum(mi)
        dst_v[pl.ds(k * 16, 16)] = dstv
        idx_v[k // (SUB // 16), pl.ds((k % (SUB // 16)) * 16, 16)] = dstv

    pltpu.sync_copy(dst_v, dst_hbm.at[pl.ds(t0, CHUNK)])

    for j in range(NSUB):
        pltpu.sync_copy(xw_hbm.at[pl.ds(t0 + j * SUB, SUB)], buf_v)
        pltpu.async_copy(buf_v, xs_hbm.at[idx_v.at[j]], sem).wait()

    @pl.when(wid == 0)
    def _():
        nblk_used = jnp.sum(jnp.where(lane == E - 1, incl, 0)) >> BLK_SH
        for j in range(NMAP // 16):
            pos = (lax.iota(jnp.int32, 16) + j * 16) * BLK
            cnt = jnp.zeros((16,), jnp.int32)
            for e in range(1, E):
                st = jnp.sum(jnp.where(lane == e, seg_start, 0))
                cnt = cnt + (pos >= st).astype(jnp.int32)
            if j == NMAP // 16 - 1:
                cnt = jnp.where(lax.iota(jnp.int32, 16) == 15, nblk_used, cnt)
            map_v[pl.ds(j * 16, 16)] = cnt
        pltpu.sync_copy(map_v, map_hbm)


NQ = 4
QF = D_FF // NQ


def _w1_copy(w1_hbm, w1f, s1, ex, q):
    return pltpu.make_async_copy(
        w1_hbm.at[ex, :, pl.ds(q * QF, QF)],
        w1f.at[:, pl.ds(q * QF, QF)], s1.at[q])


def _w2_copy(w2_hbm, w2f, s2, ex, q):
    return pltpu.make_async_copy(
        w2_hbm.at[ex, pl.ds(q * QF, QF), :],
        w2f.at[pl.ds(q * QF, QF), :], s2.at[q])


def _ffn_body(map_ref, xs_ref, w1_hbm, b1_ref, w2_hbm, b2_ref, ys_ref,
              w1f, w2f, st, s1, s2):
    b = pl.program_id(0)
    e = map_ref[b]
    nb = map_ref[NMAP - 1]
    live = b < nb
    is_sw = jnp.logical_and(live, jnp.logical_or(b == 0, e != st[0]))

    @pl.when(jnp.logical_and(is_sw, b == 0))
    def _():
        for q in range(NQ):
            _w1_copy(w1_hbm, w1f, s1, e, q).start()
            _w2_copy(w2_hbm, w2f, s2, e, q).start()

    @pl.when(is_sw)
    def _():
        st[0] = e

    @pl.when(live)
    def _():
        xb = xs_ref[...]
        y = jnp.zeros((BLK, D), jnp.float32)
        for q in range(NQ):
            @pl.when(is_sw)
            def _():
                _w1_copy(w1_hbm, w1f, s1, e, q).wait()

            hq = jnp.dot(xb, w1f[:, q * QF:(q + 1) * QF],
                         preferred_element_type=jnp.float32)
            hq = jnp.maximum(hq + b1_ref[0, :, q * QF:(q + 1) * QF],
                             0.0).astype(jnp.bfloat16)

            @pl.when(is_sw)
            def _():
                _w2_copy(w2_hbm, w2f, s2, e, q).wait()

            y = y + jnp.dot(hq, w2f[q * QF:(q + 1) * QF, :],
                            preferred_element_type=jnp.float32)
        ys_ref[...] = y + b2_ref[0]

    en1 = map_ref[b + 1]
    do_pref = live & (en1 != e) & ((b + 1) < nb)

    @pl.when(do_pref)
    def _():
        for q in range(NQ):
            _w1_copy(w1_hbm, w1f, s1, en1, q).start()
            _w2_copy(w2_hbm, w2f, s2, en1, q).start()


def _ffn(bmap, xs, W1, b1, W2, b2):
    grid_spec = pltpu.PrefetchScalarGridSpec(
        num_scalar_prefetch=1,
        grid=(NBLK,),
        in_specs=[
            pl.BlockSpec((BLK, D), lambda b, m: (b, 0)),
            pl.BlockSpec(memory_space=pl.ANY),
            pl.BlockSpec((1, 1, D_FF), lambda b, m: (m[b], 0, 0)),
            pl.BlockSpec(memory_space=pl.ANY),
            pl.BlockSpec((1, 1, D), lambda b, m: (m[b], 0, 0)),
        ],
        out_specs=pl.BlockSpec((BLK, D), lambda b, m: (b, 0)),
        scratch_shapes=[
            pltpu.VMEM((D, D_FF), jnp.float32),
            pltpu.VMEM((D_FF, D), jnp.float32),
            pltpu.SMEM((4,), jnp.int32),
            pltpu.SemaphoreType.DMA((NQ,)),
            pltpu.SemaphoreType.DMA((NQ,)),
        ],
    )
    return pl.pallas_call(
        _ffn_body,
        grid_spec=grid_spec,
        out_shape=jax.ShapeDtypeStruct((NPAD, D), jnp.float32),
        compiler_params=pltpu.CompilerParams(
            dimension_semantics=("arbitrary",)),
    )(bmap, xs, W1, b1, W2, b2)


@functools.partial(
    pl.kernel,
    out_type=jax.ShapeDtypeStruct((N, D), jnp.float32),
    mesh=_SC_MESH,
    scratch_types=[
        pltpu.VMEM((NSUB, SUB), jnp.int32),
        pltpu.VMEM((SUB, D), jnp.float32),
        pltpu.SemaphoreType.DMA,
    ],
    compiler_params=_SC_PARAMS,
)
def _combine(ys_hbm, dst_hbm, out_hbm, idx_v, buf_v, sem):
    wid = lax.axis_index("s") * 2 + lax.axis_index("c")
    t0 = wid * CHUNK
    for j in range(NSUB):
        pltpu.sync_copy(dst_hbm.at[pl.ds(t0 + j * SUB, SUB)], idx_v.at[j])
    for j in range(NSUB):
        pltpu.async_copy(ys_hbm.at[idx_v.at[j]], buf_v, sem).wait()
        pltpu.sync_copy(buf_v, out_hbm.at[pl.ds(t0 + j * SUB, SUB)])


def kernel(x, Wg, bg, W1, b1, W2, b2):
    xf = x.reshape(N, D)
    xw, e2, cnt3 = _router(xf, Wg, bg)
    xs, dst, bmap = _dispatch(e2.reshape(N), cnt3.reshape(NW, 16), xw)
    ys = _ffn(bmap, xs, W1, b1.reshape(E, 1, D_FF), W2, b2.reshape(E, 1, D))
    out = _combine(ys, dst)
    return out.reshape(B, S, D)


# speedup vs baseline: 1.1121x; 1.0216x over previous
import dataclasses
import functools

import jax
import jax.numpy as jnp
from jax import lax
from jax.experimental import pallas as pl
from jax.experimental.pallas import tpu as pltpu
from jax.experimental.pallas import tpu_sc as plsc

B, S, D, E, TOP_K = 4, 2048, 1024, 8, 2
D_FF = 4 * D
N = B * S
BLK = 256
BLK_SH = 8
NW = 32
CHUNK = N // NW
NBLK = N // BLK + E
NPAD = NBLK * BLK
NMAP = ((NBLK + 16) // 16) * 16
SUB = 64
NSUB = CHUNK // SUB


BLKR = 1024
NCH_R = BLKR // CHUNK


def _router_body(x_ref, wg_ref, bg_ref, xw_ref, e_ref, cnt_ref):
    xb = x_ref[...]
    logits = jnp.dot(xb, wg_ref[...], preferred_element_type=jnp.float32)
    logits = logits + bg_ref[...]
    m = jnp.max(logits, axis=-1, keepdims=True)
    ex = jnp.exp(logits - m)
    s = ex / jnp.sum(ex, axis=-1, keepdims=True)
    lane = lax.broadcasted_iota(jnp.int32, (BLKR, E), 1)
    m1 = jnp.max(s, axis=-1, keepdims=True)
    i1 = jnp.min(jnp.where(s == m1, lane, E), axis=-1, keepdims=True)
    s2 = jnp.where(lane == i1, -jnp.inf, s)
    m2 = jnp.max(s2, axis=-1, keepdims=True)
    i2 = jnp.min(jnp.where(s2 == m2, lane, E), axis=-1, keepdims=True)
    estar = jnp.maximum(i1, i2)
    w = jnp.sum(jnp.where(lane == estar, s, 0.0), axis=-1, keepdims=True)
    xw_ref[...] = xb * w
    e_ref[...] = estar
    lane16 = lax.broadcasted_iota(jnp.int32, (BLKR, 16), 1)
    oh = (lane16 == estar).astype(jnp.int32)
    cnt_ref[...] = jnp.sum(oh.reshape(NCH_R, CHUNK, 16), axis=1,
                           keepdims=False).reshape(1, NCH_R, 16)


def _router(xf, Wg, bg):
    return pl.pallas_call(
        _router_body,
        grid=(N // BLKR,),
        in_specs=[
            pl.BlockSpec((BLKR, D), lambda b: (b, 0)),
            pl.BlockSpec((D, E), lambda b: (0, 0)),
            pl.BlockSpec((1, E), lambda b: (0, 0)),
        ],
        out_specs=[
            pl.BlockSpec((BLKR, D), lambda b: (b, 0)),
            pl.BlockSpec((BLKR, 1), lambda b: (b, 0)),
            pl.BlockSpec((1, NCH_R, 16), lambda b: (b, 0, 0)),
        ],
        out_shape=[
            jax.ShapeDtypeStruct((N, D), jnp.float32),
            jax.ShapeDtypeStruct((N, 1), jnp.int32),
            jax.ShapeDtypeStruct((N // BLKR, NCH_R, 16), jnp.int32),
        ],
    )(xf, Wg, bg.reshape(1, E))


_SC_MESH = plsc.VectorSubcoreMesh(
    core_axis_name="c", subcore_axis_name="s", num_cores=2, num_subcores=16)

_SC_PARAMS = pltpu.CompilerParams()
if "needs_layout_passes" in pltpu.CompilerParams.__dataclass_fields__:
    _SC_PARAMS = dataclasses.replace(_SC_PARAMS, needs_layout_passes=False)


@functools.partial(
    pl.kernel,
    out_type=[
        jax.ShapeDtypeStruct((NPAD, D), jnp.float32),
        jax.ShapeDtypeStruct((N,), jnp.int32),
        jax.ShapeDtypeStruct((NMAP,), jnp.int32),
    ],
    mesh=_SC_MESH,
    scratch_types=[
        pltpu.VMEM((NW, 16), jnp.int32),
        pltpu.VMEM((CHUNK,), jnp.int32),
        pltpu.VMEM((CHUNK,), jnp.int32),
        pltpu.VMEM((NSUB, SUB), jnp.int32),
        pltpu.VMEM((SUB, D), jnp.float32),
        pltpu.VMEM((NMAP,), jnp.int32),
        pltpu.SMEM((E,), jnp.int32),
        pltpu.SemaphoreType.DMA,
    ],
    compiler_params=_SC_PARAMS,
)
def _dispatch(e_hbm, cnt_hbm, xw_hbm, xs_hbm, dst_hbm, map_hbm,
              cnt_v, e_v, dst_v, idx_v, buf_v, map_v, base_s, sem):
    wid = lax.axis_index("s") * 2 + lax.axis_index("c")
    t0 = wid * CHUNK
    pltpu.sync_copy(cnt_hbm, cnt_v)
    pltpu.sync_copy(e_hbm.at[pl.ds(t0, CHUNK)], e_v)

    lane = lax.iota(jnp.int32, 16)
    total = jnp.zeros((16,), jnp.int32)
    pref = jnp.zeros((16,), jnp.int32)
    for wp in range(NW):
        row = cnt_v[wp]
        total = total + row
        pref = pref + jnp.where(wp < wid, row, 0)
    rounded = ((total + (BLK - 1)) >> BLK_SH) << BLK_SH
    rounded = jnp.where(lane < E, rounded, 0)
    incl = plsc.cumsum(rounded)
    seg_start = incl - rounded
    my_base = seg_start + pref

    for e in range(E):
        base_s[e] = jnp.sum(jnp.where(lane == e, my_base, 0))

    for k in range(CHUNK // 16):
        ev = e_v[pl.ds(k * 16, 16)]
        dstv = jnp.zeros((16,), jnp.int32)
        for e in range(E):
            mi = (ev == e).astype(jnp.int32)
            ranks = plsc.cumsum(mi) - 1
            b = base_s[e]
            dstv = jnp.where(ev == e, b + ranks, dstv)
            base_s[e] = b + jnp.sum(mi)
        dst_v[pl.ds(k * 16, 16)] = dstv
        idx_v[k // (SUB // 16), pl.ds((k % (SUB // 16)) * 16, 16)] = dstv

    pltpu.sync_copy(dst_v, dst_hbm.at[pl.ds(t0, CHUNK)])

    for j in range(NSUB):
        pltpu.sync_copy(xw_hbm.at[pl.ds(t0 + j * SUB, SUB)], buf_v)
        pltpu.async_copy(buf_v, xs_hbm.at[idx_v.at[j]], sem).wait()

    @pl.when(wid == 0)
    def _():
        nblk_used = jnp.sum(jnp.where(lane == E - 1, incl, 0)) >> BLK_SH
        for j in range(NMAP // 16):
            pos = (lax.iota(jnp.int32, 16) + j * 16) * BLK
            cnt = jnp.zeros((16,), jnp.int32)
            for e in range(1, E):
                st = jnp.sum(jnp.where(lane == e, seg_start, 0))
                cnt = cnt + (pos >= st).astype(jnp.int32)
            if j == NMAP // 16 - 1:
                cnt = jnp.where(lax.iota(jnp.int32, 16) == 15, nblk_used, cnt)
            map_v[pl.ds(j * 16, 16)] = cnt
        pltpu.sync_copy(map_v, map_hbm)


HF = D_FF // 2


def _w1_copy(w1_hbm, w1f, s1, ex):
    return pltpu.make_async_copy(w1_hbm.at[ex], w1f, s1)


def _w2_copy(w2_hbm, w2f, s2, ex, hh):
    return pltpu.make_async_copy(
        w2_hbm.at[ex, pl.ds(hh * HF, HF), :],
        w2f.at[pl.ds(hh * HF, HF), :], s2.at[hh])


def _ffn_body(map_ref, xs_ref, w1_hbm, b1_ref, w2_hbm, b2_ref, ys_ref,
              w1f, w2f, st, s1, s2):
    b = pl.program_id(0)
    e = map_ref[b]
    nb = map_ref[NMAP - 1]
    live = b < nb
    is_sw = jnp.logical_and(live, jnp.logical_or(b == 0, e != st[0]))

    @pl.when(jnp.logical_and(is_sw, b == 0))
    def _():
        _w1_copy(w1_hbm, w1f, s1, e).start()
        _w2_copy(w2_hbm, w2f, s2, e, 0).start()
        _w2_copy(w2_hbm, w2f, s2, e, 1).start()

    @pl.when(is_sw)
    def _():
        _w1_copy(w1_hbm, w1f, s1, e).wait()
        st[0] = e

    @pl.when(live)
    def _():
        xb = xs_ref[...]
        h = jnp.dot(xb, w1f[...], preferred_element_type=jnp.float32)
        h = jnp.maximum(h + b1_ref[0], 0.0).astype(jnp.bfloat16)

        @pl.when(is_sw)
        def _():
            _w2_copy(w2_hbm, w2f, s2, e, 0).wait()

        y = jnp.dot(h[:, :HF], w2f[:HF, :], preferred_element_type=jnp.float32)

        @pl.when(is_sw)
        def _():
            _w2_copy(w2_hbm, w2f, s2, e, 1).wait()

        y = y + jnp.dot(h[:, HF:], w2f[HF:, :],
                        preferred_element_type=jnp.float32)
        ys_ref[...] = y + b2_ref[0]

    en1 = map_ref[b + 1]
    do_pref = live & (en1 != e) & ((b + 1) < nb)

    @pl.when(do_pref)
    def _():
        _w1_copy(w1_hbm, w1f, s1, en1).start()
        _w2_copy(w2_hbm, w2f, s2, en1, 0).start()
        _w2_copy(w2_hbm, w2f, s2, en1, 1).start()


def _ffn(bmap, xs, W1, b1, W2, b2):
    grid_spec = pltpu.PrefetchScalarGridSpec(
        num_scalar_prefetch=1,
        grid=(NBLK,),
        in_specs=[
            pl.BlockSpec((BLK, D), lambda b, m: (b, 0)),
            pl.BlockSpec(memory_space=pl.ANY),
            pl.BlockSpec((1, 1, D_FF), lambda b, m: (m[b], 0, 0)),
            pl.BlockSpec(memory_space=pl.ANY),
            pl.BlockSpec((1, 1, D), lambda b, m: (m[b], 0, 0)),
        ],
        out_specs=pl.BlockSpec((BLK, D), lambda b, m: (b, 0)),
        scratch_shapes=[
            pltpu.VMEM((D, D_FF), jnp.float32),
            pltpu.VMEM((D_FF, D), jnp.float32),
            pltpu.SMEM((4,), jnp.int32),
            pltpu.SemaphoreType.DMA,
            pltpu.SemaphoreType.DMA((2,)),
        ],
    )
    return pl.pallas_call(
        _ffn_body,
        grid_spec=grid_spec,
        out_shape=jax.ShapeDtypeStruct((NPAD, D), jnp.float32),
        compiler_params=pltpu.CompilerParams(
            dimension_semantics=("arbitrary",)),
    )(bmap, xs, W1, b1, W2, b2)


@functools.partial(
    pl.kernel,
    out_type=jax.ShapeDtypeStruct((N, D), jnp.float32),
    mesh=_SC_MESH,
    scratch_types=[
        pltpu.VMEM((NSUB, SUB), jnp.int32),
        pltpu.VMEM((SUB, D), jnp.float32),
        pltpu.SemaphoreType.DMA,
    ],
    compiler_params=_SC_PARAMS,
)
def _combine(ys_hbm, dst_hbm, out_hbm, idx_v, buf_v, sem):
    wid = lax.axis_index("s") * 2 + lax.axis_index("c")
    t0 = wid * CHUNK
    for j in range(NSUB):
        pltpu.sync_copy(dst_hbm.at[pl.ds(t0 + j * SUB, SUB)], idx_v.at[j])
    for j in range(NSUB):
        pltpu.async_copy(ys_hbm.at[idx_v.at[j]], buf_v, sem).wait()
        pltpu.sync_copy(buf_v, out_hbm.at[pl.ds(t0 + j * SUB, SUB)])


def kernel(x, Wg, bg, W1, b1, W2, b2):
    xf = x.reshape(N, D)
    xw, e2, cnt3 = _router(xf, Wg, bg)
    xs, dst, bmap = _dispatch(e2.reshape(N), cnt3.reshape(NW, 16), xw)
    ys = _ffn(bmap, xs, W1, b1.reshape(E, 1, D_FF), W2, b2.reshape(E, 1, D))
    out = _combine(ys, dst)
    return out.reshape(B, S, D)


# speedup vs baseline: 1.1690x; 1.0511x over previous
import dataclasses
import functools

import jax
import jax.numpy as jnp
from jax import lax
from jax.experimental import pallas as pl
from jax.experimental.pallas import tpu as pltpu
from jax.experimental.pallas import tpu_sc as plsc

B, S, D, E, TOP_K = 4, 2048, 1024, 8, 2
D_FF = 4 * D
N = B * S
BLK = 256
BLK_SH = 8
NW = 32
CHUNK = N // NW
NBLK = N // BLK + E
NPAD = NBLK * BLK
NMAP = ((NBLK + 16) // 16) * 16
SUB = 64
NSUB = CHUNK // SUB


BLKR = 1024
NCH_R = BLKR // CHUNK


def _router_body(x_ref, wg_ref, bg_ref, xw_ref, e_ref, cnt_ref):
    xb = x_ref[...]
    logits = jnp.dot(xb, wg_ref[...], preferred_element_type=jnp.float32)
    logits = logits + bg_ref[...]
    m = jnp.max(logits, axis=-1, keepdims=True)
    ex = jnp.exp(logits - m)
    s = ex / jnp.sum(ex, axis=-1, keepdims=True)
    lane = lax.broadcasted_iota(jnp.int32, (BLKR, E), 1)
    m1 = jnp.max(s, axis=-1, keepdims=True)
    i1 = jnp.min(jnp.where(s == m1, lane, E), axis=-1, keepdims=True)
    s2 = jnp.where(lane == i1, -jnp.inf, s)
    m2 = jnp.max(s2, axis=-1, keepdims=True)
    i2 = jnp.min(jnp.where(s2 == m2, lane, E), axis=-1, keepdims=True)
    estar = jnp.maximum(i1, i2)
    w = jnp.sum(jnp.where(lane == estar, s, 0.0), axis=-1, keepdims=True)
    xw_ref[...] = xb * w
    e_ref[...] = estar
    lane16 = lax.broadcasted_iota(jnp.int32, (BLKR, 16), 1)
    oh = (lane16 == estar).astype(jnp.int32)
    cnt_ref[...] = jnp.sum(oh.reshape(NCH_R, CHUNK, 16), axis=1,
                           keepdims=False).reshape(1, NCH_R, 16)


def _router(xf, Wg, bg):
    return pl.pallas_call(
        _router_body,
        grid=(N // BLKR,),
        in_specs=[
            pl.BlockSpec((BLKR, D), lambda b: (b, 0)),
            pl.BlockSpec((D, E), lambda b: (0, 0)),
            pl.BlockSpec((1, E), lambda b: (0, 0)),
        ],
        out_specs=[
            pl.BlockSpec((BLKR, D), lambda b: (b, 0)),
            pl.BlockSpec((BLKR, 1), lambda b: (b, 0)),
            pl.BlockSpec((1, NCH_R, 16), lambda b: (b, 0, 0)),
        ],
        out_shape=[
            jax.ShapeDtypeStruct((N, D), jnp.float32),
            jax.ShapeDtypeStruct((N, 1), jnp.int32),
            jax.ShapeDtypeStruct((N // BLKR, NCH_R, 16), jnp.int32),
        ],
    )(xf, Wg, bg.reshape(1, E))


_SC_MESH = plsc.VectorSubcoreMesh(
    core_axis_name="c", subcore_axis_name="s", num_cores=2, num_subcores=16)

_SC_PARAMS = pltpu.CompilerParams()
if "needs_layout_passes" in pltpu.CompilerParams.__dataclass_fields__:
    _SC_PARAMS = dataclasses.replace(_SC_PARAMS, needs_layout_passes=False)


@functools.partial(
    pl.kernel,
    out_type=[
        jax.ShapeDtypeStruct((NPAD, D), jnp.float32),
        jax.ShapeDtypeStruct((N,), jnp.int32),
        jax.ShapeDtypeStruct((NMAP,), jnp.int32),
    ],
    mesh=_SC_MESH,
    scratch_types=[
        pltpu.VMEM((NW, 16), jnp.int32),
        pltpu.VMEM((CHUNK,), jnp.int32),
        pltpu.VMEM((CHUNK,), jnp.int32),
        pltpu.VMEM((NSUB, SUB), jnp.int32),
        pltpu.VMEM((SUB, D), jnp.float32),
        pltpu.VMEM((NMAP,), jnp.int32),
        pltpu.SMEM((E,), jnp.int32),
        pltpu.SemaphoreType.DMA,
    ],
    compiler_params=_SC_PARAMS,
)
def _dispatch(e_hbm, cnt_hbm, xw_hbm, xs_hbm, dst_hbm, map_hbm,
              cnt_v, e_v, dst_v, idx_v, buf_v, map_v, base_s, sem):
    wid = lax.axis_index("s") * 2 + lax.axis_index("c")
    t0 = wid * CHUNK
    pltpu.sync_copy(cnt_hbm, cnt_v)
    pltpu.sync_copy(e_hbm.at[pl.ds(t0, CHUNK)], e_v)

    lane = lax.iota(jnp.int32, 16)
    total = jnp.zeros((16,), jnp.int32)
    pref = jnp.zeros((16,), jnp.int32)
    for wp in range(NW):
        row = cnt_v[wp]
        total = total + row
        pref = pref + jnp.where(wp < wid, row, 0)
    rounded = ((total + (BLK - 1)) >> BLK_SH) << BLK_SH
    rounded = jnp.where(lane < E, rounded, 0)
    incl = plsc.cumsum(rounded)
    seg_start = incl - rounded
    my_base = seg_start + pref

    for e in range(E):
        base_s[e] = jnp.sum(jnp.where(lane == e, my_base, 0))

    for k in range(CHUNK // 16):
        ev = e_v[pl.ds(k * 16, 16)]
        dstv = jnp.zeros((16,), jnp.int32)
        for e in range(E):
            mi = (ev == e).astype(jnp.int32)
            ranks = plsc.cumsum(mi) - 1
            b = base_s[e]
            dstv = jnp.where(ev == e, b + ranks, dstv)
            base_s[e] = b + jnp.sum(mi)
        dst_v[pl.ds(k * 16, 16)] = dstv
        idx_v[k // (SUB // 16), pl.ds((k % (SUB // 16)) * 16, 16)] = dstv

    pltpu.sync_copy(dst_v, dst_hbm.at[pl.ds(t0, CHUNK)])

    for j in range(NSUB):
        pltpu.sync_copy(xw_hbm.at[pl.ds(t0 + j * SUB, SUB)], buf_v)
        pltpu.async_copy(buf_v, xs_hbm.at[idx_v.at[j]], sem).wait()

    @pl.when(wid == 0)
    def _():
        nblk_used = jnp.sum(jnp.where(lane == E - 1, incl, 0)) >> BLK_SH
        for j in range(NMAP // 16):
            pos = (lax.iota(jnp.int32, 16) + j * 16) * BLK
            cnt = jnp.zeros((16,), jnp.int32)
            for e in range(1, E):
                st = jnp.sum(jnp.where(lane == e, seg_start, 0))
                cnt = cnt + (pos >= st).astype(jnp.int32)
            if j == NMAP // 16 - 1:
                cnt = jnp.where(lax.iota(jnp.int32, 16) == 15, nblk_used, cnt)
            map_v[pl.ds(j * 16, 16)] = cnt
        pltpu.sync_copy(map_v, map_hbm)


def _w1_copy(w1_hbm, w1f, s1, ex):
    return pltpu.make_async_copy(w1_hbm.at[ex], w1f, s1)


def _w2_copy(w2_hbm, w2f, s2, ex):
    return pltpu.make_async_copy(w2_hbm.at[ex], w2f, s2)


def _ffn_body(map_ref, xs_ref, w1_hbm, b1_ref, w2_hbm, b2_ref, ys_ref,
              w1f, w2f, st, s1, s2):
    b = pl.program_id(0)
    e = map_ref[b]
    nb = map_ref[NMAP - 1]
    live = b < nb
    is_sw = jnp.logical_and(live, jnp.logical_or(b == 0, e != st[0]))

    @pl.when(jnp.logical_and(is_sw, b == 0))
    def _():
        _w1_copy(w1_hbm, w1f, s1, e).start()
        _w2_copy(w2_hbm, w2f, s2, e).start()

    @pl.when(is_sw)
    def _():
        _w1_copy(w1_hbm, w1f, s1, e).wait()
        st[0] = e

    @pl.when(live)
    def _():
        xb = xs_ref[...]
        h = jnp.dot(xb, w1f[...], preferred_element_type=jnp.float32)
        h = jnp.maximum(h + b1_ref[0], 0.0)

        @pl.when(is_sw)
        def _():
            _w2_copy(w2_hbm, w2f, s2, e).wait()

        y = jnp.dot(h, w2f[...], preferred_element_type=jnp.float32)
        ys_ref[...] = y + b2_ref[0]

    en1 = map_ref[b + 1]
    do_pref = live & (en1 != e) & ((b + 1) < nb)

    @pl.when(do_pref)
    def _():
        _w1_copy(w1_hbm, w1f, s1, en1).start()
        _w2_copy(w2_hbm, w2f, s2, en1).start()


def _ffn(bmap, xs, W1, b1, W2, b2):
    grid_spec = pltpu.PrefetchScalarGridSpec(
        num_scalar_prefetch=1,
        grid=(NBLK,),
        in_specs=[
            pl.BlockSpec((BLK, D), lambda b, m: (b, 0)),
            pl.BlockSpec(memory_space=pl.ANY),
            pl.BlockSpec((1, 1, D_FF), lambda b, m: (m[b], 0, 0)),
            pl.BlockSpec(memory_space=pl.ANY),
            pl.BlockSpec((1, 1, D), lambda b, m: (m[b], 0, 0)),
        ],
        out_specs=pl.BlockSpec((BLK, D), lambda b, m: (b, 0)),
        scratch_shapes=[
            pltpu.VMEM((D, D_FF), jnp.float32),
            pltpu.VMEM((D_FF, D), jnp.float32),
            pltpu.SMEM((4,), jnp.int32),
            pltpu.SemaphoreType.DMA,
            pltpu.SemaphoreType.DMA,
        ],
    )
    return pl.pallas_call(
        _ffn_body,
        grid_spec=grid_spec,
        out_shape=jax.ShapeDtypeStruct((NPAD, D), jnp.float32),
        compiler_params=pltpu.CompilerParams(
            dimension_semantics=("arbitrary",)),
    )(bmap, xs, W1, b1, W2, b2)


@functools.partial(
    pl.kernel,
    out_type=jax.ShapeDtypeStruct((N, D), jnp.float32),
    mesh=_SC_MESH,
    scratch_types=[
        pltpu.VMEM((NSUB, SUB), jnp.int32),
        pltpu.VMEM((SUB, D), jnp.float32),
        pltpu.SemaphoreType.DMA,
    ],
    compiler_params=_SC_PARAMS,
)
def _combine(ys_hbm, dst_hbm, out_hbm, idx_v, buf_v, sem):
    wid = lax.axis_index("s") * 2 + lax.axis_index("c")
    t0 = wid * CHUNK
    for j in range(NSUB):
        pltpu.sync_copy(dst_hbm.at[pl.ds(t0 + j * SUB, SUB)], idx_v.at[j])
    for j in range(NSUB):
        pltpu.async_copy(ys_hbm.at[idx_v.at[j]], buf_v, sem).wait()
        pltpu.sync_copy(buf_v, out_hbm.at[pl.ds(t0 + j * SUB, SUB)])


def kernel(x, Wg, bg, W1, b1, W2, b2):
    xf = x.reshape(N, D)
    xw, e2, cnt3 = _router(xf, Wg, bg)
    xs, dst, bmap = _dispatch(e2.reshape(N), cnt3.reshape(NW, 16), xw)
    ys = _ffn(bmap, xs, W1, b1.reshape(E, 1, D_FF), W2, b2.reshape(E, 1, D))
    out = _combine(ys, dst)
    return out.reshape(B, S, D)
